# baseline (speedup 1.0000x reference)
import jax
import jax.numpy as jnp
from jax import lax
from jax.experimental import pallas as pl
from jax.experimental.pallas import tpu as pltpu
from jax.experimental.pallas import tpu_sc as plsc

_NC = 2
_NS = 16
_NW = _NC * _NS

_CHUNK = 1280


def _emb_body(idx_hbm, table_hbm, out_hbm, idx_v, rows_v, gsem0, gsem1, osem0, osem1):
    n = idx_hbm.shape[0]
    b_per_w = n // _NW
    nblk = b_per_w // _CHUNK
    npair = nblk // 2
    wid = lax.axis_index("s") * _NC + lax.axis_index("c")
    base0 = wid * b_per_w
    gsems = (gsem0, gsem1)
    osems = (osem0, osem1)

    def fill(s, base):
        pltpu.sync_copy(idx_hbm.at[pl.ds(base, _CHUNK)], idx_v.at[s])
        return pltpu.async_copy(table_hbm.at[idx_v.at[s]], rows_v.at[s], gsems[s])

    def flush(s, base, copy):
        copy.wait()
        pltpu.async_copy(rows_v.at[s], out_hbm.at[pl.ds(base, _CHUNK)], osems[s])

    def out_wait(s, base):
        pltpu.make_async_copy(rows_v.at[s], out_hbm.at[pl.ds(base, _CHUNK)], osems[s]).wait()

    for s in range(2):
        flush(s, base0 + s * _CHUNK, fill(s, base0 + s * _CHUNK))

    def body(pair, carry):
        b0 = base0 + 2 * pair * _CHUNK
        cs = []
        for s in range(2):
            out_wait(s, b0 + s * _CHUNK)
            cs.append(fill(s, b0 + s * _CHUNK))
        for s in range(2):
            flush(s, b0 + s * _CHUNK, cs[s])
        return carry

    lax.fori_loop(1, npair, body, 0)
    for s in range(2):
        out_wait(s, base0 + s * _CHUNK)


def kernel(input, embedding_weight):
    B, L = input.shape
    V, D = embedding_weight.shape
    n = B * L
    idx_flat = input.reshape(n)
    table_bf = embedding_weight.astype(jnp.bfloat16)
    mesh = plsc.VectorSubcoreMesh(core_axis_name="c", subcore_axis_name="s")
    run = pl.kernel(
        _emb_body,
        mesh=mesh,
        out_type=jax.ShapeDtypeStruct((n, D), jnp.bfloat16),
        scratch_types=[
            pltpu.VMEM((2, _CHUNK), jnp.int32),
            pltpu.VMEM((2, _CHUNK, D), jnp.bfloat16),
            pltpu.SemaphoreType.DMA,
            pltpu.SemaphoreType.DMA,
            pltpu.SemaphoreType.DMA,
            pltpu.SemaphoreType.DMA,
        ],
        compiler_params=pltpu.CompilerParams(use_tc_tiling_on_sc=False),
    )
    out = run(idx_flat, table_bf)
    return out.astype(jnp.float32).reshape(B, L, D)


# speedup vs baseline: 1.1152x; 1.1152x over previous
import jax
import jax.numpy as jnp
from jax import lax
from jax.experimental import pallas as pl
from jax.experimental.pallas import tpu as pltpu
from jax.experimental.pallas import tpu_sc as plsc

_NC = 2
_NS = 16
_NW = _NC * _NS

_CHUNK = 1600


def _emb_body(idx_hbm, table_hbm, out_hbm, idx_v, rows_v, gsem0, gsem1, osem0, osem1):
    n = idx_hbm.shape[0]
    b_per_w = n // _NW
    nblk = b_per_w // _CHUNK
    npair = nblk // 2
    wid = lax.axis_index("s") * _NC + lax.axis_index("c")
    base0 = wid * b_per_w
    gsems = (gsem0, gsem1)
    osems = (osem0, osem1)

    pltpu.sync_copy(idx_hbm.at[pl.ds(base0, b_per_w)], idx_v)

    def fill(s, blk):
        return pltpu.async_copy(
            table_hbm.at[idx_v.at[pl.ds(blk * _CHUNK, _CHUNK)]],
            rows_v.at[s],
            gsems[s],
        )

    def flush(s, blk, copy):
        copy.wait()
        pltpu.async_copy(
            rows_v.at[s], out_hbm.at[pl.ds(base0 + blk * _CHUNK, _CHUNK)], osems[s]
        )

    def out_wait(s, blk):
        pltpu.make_async_copy(
            rows_v.at[s], out_hbm.at[pl.ds(base0 + blk * _CHUNK, _CHUNK)], osems[s]
        ).wait()

    for s in range(2):
        flush(s, s, fill(s, s))

    def body(pair, carry):
        cs = []
        for s in range(2):
            out_wait(s, 2 * pair + s)
            cs.append(fill(s, 2 * pair + s))
        for s in range(2):
            flush(s, 2 * pair + s, cs[s])
        return carry

    lax.fori_loop(1, npair, body, 0)
    for s in range(2):
        out_wait(s, s)


def kernel(input, embedding_weight):
    B, L = input.shape
    V, D = embedding_weight.shape
    n = B * L
    idx_flat = input.reshape(n)
    mesh = plsc.VectorSubcoreMesh(core_axis_name="c", subcore_axis_name="s")
    run = pl.kernel(
        _emb_body,
        mesh=mesh,
        out_type=jax.ShapeDtypeStruct((n, D), jnp.float32),
        scratch_types=[
            pltpu.VMEM((n // _NW,), jnp.int32),
            pltpu.VMEM((2, _CHUNK, D), jnp.float32),
            pltpu.SemaphoreType.DMA,
            pltpu.SemaphoreType.DMA,
            pltpu.SemaphoreType.DMA,
            pltpu.SemaphoreType.DMA,
        ],
        compiler_params=pltpu.CompilerParams(use_tc_tiling_on_sc=False),
    )
    out = run(idx_flat, embedding_weight)
    return out.reshape(B, L, D)
